# Initial kernel scaffold; baseline (speedup 1.0000x reference)
import functools

import jax
import jax.numpy as jnp
from jax import lax
from jax.experimental import pallas as pl
from jax.experimental.pallas import tpu as pltpu
from jax.experimental.pallas import tpu_sc as plsc

_N = 10000
_E = 320000
_F_IN = 128
_H = 64
_G = 64
_C = 16

_NC = 2
_NS = 16
_NW = _NC * _NS
_CH = 128
_NCHUNK = 79
_EPAD = _NW * _CH * _NCHUNK
_ACC_ROWS = 10240
_ROWS_PER_SUB = _ACC_ROWS // _NS
_OUT_ROWS_PER_SUB = _N // _NS
_ZCOPIES = _ROWS_PER_SUB // _CH

_RB = 1000
_NBLK = _N // _RB


def _sc_segment_sum(y, src3, dst3):
    mesh = plsc.VectorSubcoreMesh(core_axis_name="c", subcore_axis_name="s")

    @functools.partial(
        pl.kernel,
        out_type=jax.ShapeDtypeStruct((_NC, _N, _H), jnp.float32),
        mesh=mesh,
        scratch_types=[
            pltpu.VMEM((_NCHUNK, _CH), jnp.int32),
            pltpu.VMEM((_NCHUNK, _CH), jnp.int32),
            pltpu.VMEM((_CH, _H), jnp.float32),
            pltpu.VMEM_SHARED((_ACC_ROWS, _H), jnp.float32),
            pltpu.SemaphoreType.DMA,
        ],
    )
    def k(y_hbm, src_hbm, dst_hbm, out_hbm, src_v, dst_v, rows_v, acc_sh, sem):
        cid = lax.axis_index("c")
        sid = lax.axis_index("s")
        wid = sid * _NC + cid

        zeros = jnp.zeros((16,), jnp.float32)

        def zrow(r, carry):
            for cc in range(_H // 16):
                rows_v[r, pl.ds(cc * 16, 16)] = zeros
            return carry

        lax.fori_loop(0, _CH, zrow, 0)
        for b in range(_ZCOPIES):
            pltpu.sync_copy(
                rows_v, acc_sh.at[pl.ds(sid * _ROWS_PER_SUB + b * _CH, _CH)]
            )

        pltpu.sync_copy(src_hbm.at[wid], src_v)
        pltpu.sync_copy(dst_hbm.at[wid], dst_v)
        plsc.subcore_barrier()

        def body(j, carry):
            pltpu.async_copy(y_hbm.at[src_v.at[j]], rows_v, sem).wait()
            pltpu.sync_copy(rows_v, acc_sh.at[dst_v.at[j]], add=True)
            return carry

        lax.fori_loop(0, _NCHUNK, body, 0)

        plsc.subcore_barrier()
        pltpu.sync_copy(
            acc_sh.at[pl.ds(sid * _OUT_ROWS_PER_SUB, _OUT_ROWS_PER_SUB)],
            out_hbm.at[cid, pl.ds(sid * _OUT_ROWS_PER_SUB, _OUT_ROWS_PER_SUB)],
        )

    return k(y, src3, dst3)


def _tc_pre(x, w_rel, w_root):
    def body(x_ref, wa_ref, wb_ref, y_ref, r_ref):
        xb = x_ref[...]
        y_ref[...] = jnp.dot(xb, wa_ref[...], preferred_element_type=jnp.float32)
        r_ref[...] = jnp.dot(xb, wb_ref[...], preferred_element_type=jnp.float32)

    return pl.pallas_call(
        body,
        grid=(_NBLK,),
        in_specs=[
            pl.BlockSpec((_RB, _F_IN), lambda i: (i, 0)),
            pl.BlockSpec((_F_IN, _H), lambda i: (0, 0)),
            pl.BlockSpec((_F_IN, _H), lambda i: (0, 0)),
        ],
        out_specs=[
            pl.BlockSpec((_RB, _H), lambda i: (i, 0)),
            pl.BlockSpec((_RB, _H), lambda i: (i, 0)),
        ],
        out_shape=[
            jax.ShapeDtypeStruct((_N, _H), jnp.float32),
            jax.ShapeDtypeStruct((_N, _H), jnp.float32),
        ],
    )(x, w_rel, w_root)


def _pool_accum(i, onehot, h, pool_ref):
    psum = jnp.dot(onehot, h, preferred_element_type=jnp.float32)

    @pl.when(i == 0)
    def _():
        pool_ref[...] = psum

    @pl.when(i > 0)
    def _():
        pool_ref[...] += psum


def _tc_combine1(parts, r, b_rel, batch3):
    def body(parts_ref, r_ref, b_ref, batch_ref, h_ref, pool_ref, cnt_ref):
        i = pl.program_id(0)
        agg = parts_ref[0] + parts_ref[1]
        h = jnp.maximum(agg + b_ref[...] + r_ref[...], 0.0)
        h_ref[...] = h
        bt = batch_ref[0]
        gids = lax.broadcasted_iota(jnp.int32, (_G, _RB), 0)
        onehot = (gids == bt).astype(jnp.float32)
        _pool_accum(i, onehot, h, pool_ref)
        csum = jnp.sum(onehot, axis=1, keepdims=True)

        @pl.when(i == 0)
        def _():
            cnt_ref[...] = csum

        @pl.when(i > 0)
        def _():
            cnt_ref[...] += csum

    return pl.pallas_call(
        body,
        grid=(_NBLK,),
        in_specs=[
            pl.BlockSpec((_NC, _RB, _H), lambda i: (0, i, 0)),
            pl.BlockSpec((_RB, _H), lambda i: (i, 0)),
            pl.BlockSpec((1, _H), lambda i: (0, 0)),
            pl.BlockSpec((1, 1, _RB), lambda i: (i, 0, 0)),
        ],
        out_specs=[
            pl.BlockSpec((_RB, _H), lambda i: (i, 0)),
            pl.BlockSpec((_G, _H), lambda i: (0, 0)),
            pl.BlockSpec((_G, 1), lambda i: (0, 0)),
        ],
        out_shape=[
            jax.ShapeDtypeStruct((_N, _H), jnp.float32),
            jax.ShapeDtypeStruct((_G, _H), jnp.float32),
            jax.ShapeDtypeStruct((_G, 1), jnp.float32),
        ],
    )(parts, r, b_rel, batch3)


def _tc_combine23(parts, h_prev, w_rel, w_root, b_rel, batch3):
    def body(parts_ref, hp_ref, wa_ref, wb_ref, b_ref, batch_ref, h_ref, pool_ref):
        i = pl.program_id(0)
        agg = parts_ref[0] + parts_ref[1]
        h = jnp.maximum(
            jnp.dot(agg, wa_ref[...], preferred_element_type=jnp.float32)
            + jnp.dot(hp_ref[...], wb_ref[...], preferred_element_type=jnp.float32)
            + b_ref[...],
            0.0,
        )
        h_ref[...] = h
        bt = batch_ref[0]
        gids = lax.broadcasted_iota(jnp.int32, (_G, _RB), 0)
        onehot = (gids == bt).astype(jnp.float32)
        _pool_accum(i, onehot, h, pool_ref)

    return pl.pallas_call(
        body,
        grid=(_NBLK,),
        in_specs=[
            pl.BlockSpec((_NC, _RB, _H), lambda i: (0, i, 0)),
            pl.BlockSpec((_RB, _H), lambda i: (i, 0)),
            pl.BlockSpec((_H, _H), lambda i: (0, 0)),
            pl.BlockSpec((_H, _H), lambda i: (0, 0)),
            pl.BlockSpec((1, _H), lambda i: (0, 0)),
            pl.BlockSpec((1, 1, _RB), lambda i: (i, 0, 0)),
        ],
        out_specs=[
            pl.BlockSpec((_RB, _H), lambda i: (i, 0)),
            pl.BlockSpec((_G, _H), lambda i: (0, 0)),
        ],
        out_shape=[
            jax.ShapeDtypeStruct((_N, _H), jnp.float32),
            jax.ShapeDtypeStruct((_G, _H), jnp.float32),
        ],
    )(parts, h_prev, w_rel, w_root, b_rel, batch3)


def _tc_final(p1, p2, p3, cnt, w1, b1, w2, b2):
    def body(p1_ref, p2_ref, p3_ref, cnt_ref, w1_ref, b1_ref, w2_ref, b2_ref, o_ref):
        inv = 1.0 / jnp.maximum(cnt_ref[...], 1.0)
        j = jnp.concatenate([p1_ref[...], p2_ref[...], p3_ref[...]], axis=1) * inv
        a = jnp.maximum(
            jnp.dot(j, w1_ref[...], preferred_element_type=jnp.float32) + b1_ref[...],
            0.0,
        )
        o_ref[...] = (
            jnp.dot(a, w2_ref[...], preferred_element_type=jnp.float32) + b2_ref[...]
        )

    return pl.pallas_call(
        body,
        out_shape=jax.ShapeDtypeStruct((_G, _C), jnp.float32),
    )(p1, p2, p3, cnt, w1, b1, w2, b2)


def kernel(x, edge_index, batch, W_rel1, b_rel1, W_root1, W_rel2, b_rel2, W_root2,
           W_rel3, b_rel3, W_root3, W1, b1, W2, b2):
    src = edge_index[0].astype(jnp.int32)
    dst = edge_index[1].astype(jnp.int32)
    pad = _EPAD - _E
    src3 = jnp.concatenate([src, jnp.zeros((pad,), jnp.int32)]).reshape(
        _NW, _NCHUNK, _CH)
    dst3 = jnp.concatenate([dst, jnp.full((pad,), _N, jnp.int32)]).reshape(
        _NW, _NCHUNK, _CH)
    batch3 = batch.astype(jnp.int32).reshape(_NBLK, 1, _RB)

    y1, r1 = _tc_pre(x, W_rel1, W_root1)
    parts1 = _sc_segment_sum(y1, src3, dst3)
    h1, pool1, cnt = _tc_combine1(parts1, r1, b_rel1.reshape(1, _H), batch3)
    parts2 = _sc_segment_sum(h1, src3, dst3)
    h2, pool2 = _tc_combine23(parts2, h1, W_rel2, W_root2,
                              b_rel2.reshape(1, _H), batch3)
    parts3 = _sc_segment_sum(h2, src3, dst3)
    _h3, pool3 = _tc_combine23(parts3, h2, W_rel3, W_root3,
                               b_rel3.reshape(1, _H), batch3)
    return _tc_final(pool1, pool2, pool3, cnt, W1, b1.reshape(1, _H),
                     W2, b2.reshape(1, _C))


# speedup vs baseline: 7.1361x; 7.1361x over previous
import functools

import jax
import jax.numpy as jnp
from jax import lax
from jax.experimental import pallas as pl
from jax.experimental.pallas import tpu as pltpu
from jax.experimental.pallas import tpu_sc as plsc

_N = 10000
_E = 320000
_F_IN = 128
_H = 64
_G = 64
_C = 16

_NC = 2
_NS = 16
_NW = _NC * _NS
_CH = 128
_NCHUNK = 79
_EPAD = _NW * _CH * _NCHUNK
_ACC_ROWS = 10240
_ROWS_PER_SUB = _ACC_ROWS // _NS
_OUT_ROWS_PER_SUB = _N // _NS
_ZCOPIES = _ROWS_PER_SUB // _CH

_RB = 1000
_NBLK = _N // _RB


def _sc_segment_sum(y, src3, dst3):
    mesh = plsc.VectorSubcoreMesh(core_axis_name="c", subcore_axis_name="s")

    @functools.partial(
        pl.kernel,
        out_type=jax.ShapeDtypeStruct((_NC, _N, _H), jnp.float32),
        mesh=mesh,
        scratch_types=[
            pltpu.VMEM((_NCHUNK, _CH), jnp.int32),
            pltpu.VMEM((_NCHUNK, _CH), jnp.int32),
            pltpu.VMEM((_CH, _H), jnp.float32),
            pltpu.VMEM_SHARED((_ACC_ROWS, _H), jnp.float32),
            pltpu.SemaphoreType.DMA,
        ],
        compiler_params=pltpu.CompilerParams(use_tc_tiling_on_sc=False),
    )
    def k(y_hbm, src_hbm, dst_hbm, out_hbm, src_v, dst_v, rows_v, acc_sh, sem):
        cid = lax.axis_index("c")
        sid = lax.axis_index("s")
        wid = sid * _NC + cid

        zeros = jnp.zeros((16,), jnp.float32)

        def zrow(r, carry):
            for cc in range(_H // 16):
                rows_v[r, pl.ds(cc * 16, 16)] = zeros
            return carry

        lax.fori_loop(0, _CH, zrow, 0)
        for b in range(_ZCOPIES):
            pltpu.sync_copy(
                rows_v, acc_sh.at[pl.ds(sid * _ROWS_PER_SUB + b * _CH, _CH)]
            )

        pltpu.sync_copy(src_hbm.at[wid], src_v)
        pltpu.sync_copy(dst_hbm.at[wid], dst_v)
        plsc.subcore_barrier()

        def body(j, carry):
            pltpu.async_copy(y_hbm.at[src_v.at[j]], rows_v, sem).wait()
            pltpu.sync_copy(rows_v, acc_sh.at[dst_v.at[j]], add=True)
            return carry

        lax.fori_loop(0, _NCHUNK, body, 0)

        plsc.subcore_barrier()
        off = pl.multiple_of(sid * 624, 8)

        @pl.when(sid < _NS - 1)
        def _():
            pltpu.sync_copy(acc_sh.at[pl.ds(off, 624)],
                            out_hbm.at[cid, pl.ds(off, 624)])

        @pl.when(sid == _NS - 1)
        def _():
            pltpu.sync_copy(acc_sh.at[pl.ds(9360, 640)],
                            out_hbm.at[cid, pl.ds(9360, 640)])

    return k(y, src3, dst3)


def _tc_pre(x, w_rel, w_root):
    def body(x_ref, wa_ref, wb_ref, y_ref, r_ref):
        xb = x_ref[...]
        y_ref[...] = jnp.dot(xb, wa_ref[...], preferred_element_type=jnp.float32)
        r_ref[...] = jnp.dot(xb, wb_ref[...], preferred_element_type=jnp.float32)

    return pl.pallas_call(
        body,
        grid=(_NBLK,),
        in_specs=[
            pl.BlockSpec((_RB, _F_IN), lambda i: (i, 0)),
            pl.BlockSpec((_F_IN, _H), lambda i: (0, 0)),
            pl.BlockSpec((_F_IN, _H), lambda i: (0, 0)),
        ],
        out_specs=[
            pl.BlockSpec((_RB, _H), lambda i: (i, 0)),
            pl.BlockSpec((_RB, _H), lambda i: (i, 0)),
        ],
        out_shape=[
            jax.ShapeDtypeStruct((_N, _H), jnp.float32),
            jax.ShapeDtypeStruct((_N, _H), jnp.float32),
        ],
    )(x, w_rel, w_root)


def _pool_accum(i, onehot, h, pool_ref):
    psum = jnp.dot(onehot, h, preferred_element_type=jnp.float32)

    @pl.when(i == 0)
    def _():
        pool_ref[...] = psum

    @pl.when(i > 0)
    def _():
        pool_ref[...] += psum


def _tc_combine1(parts, r, b_rel, batch3):
    def body(parts_ref, r_ref, b_ref, batch_ref, h_ref, pool_ref, cnt_ref):
        i = pl.program_id(0)
        agg = parts_ref[0] + parts_ref[1]
        h = jnp.maximum(agg + b_ref[...] + r_ref[...], 0.0)
        h_ref[...] = h
        bt = batch_ref[0]
        gids = lax.broadcasted_iota(jnp.int32, (_G, _RB), 0)
        onehot = (gids == bt).astype(jnp.float32)
        _pool_accum(i, onehot, h, pool_ref)
        csum = jnp.sum(onehot, axis=1, keepdims=True)

        @pl.when(i == 0)
        def _():
            cnt_ref[...] = csum

        @pl.when(i > 0)
        def _():
            cnt_ref[...] += csum

    return pl.pallas_call(
        body,
        grid=(_NBLK,),
        in_specs=[
            pl.BlockSpec((_NC, _RB, _H), lambda i: (0, i, 0)),
            pl.BlockSpec((_RB, _H), lambda i: (i, 0)),
            pl.BlockSpec((1, _H), lambda i: (0, 0)),
            pl.BlockSpec((1, 1, _RB), lambda i: (i, 0, 0)),
        ],
        out_specs=[
            pl.BlockSpec((_RB, _H), lambda i: (i, 0)),
            pl.BlockSpec((_G, _H), lambda i: (0, 0)),
            pl.BlockSpec((_G, 1), lambda i: (0, 0)),
        ],
        out_shape=[
            jax.ShapeDtypeStruct((_N, _H), jnp.float32),
            jax.ShapeDtypeStruct((_G, _H), jnp.float32),
            jax.ShapeDtypeStruct((_G, 1), jnp.float32),
        ],
    )(parts, r, b_rel, batch3)


def _tc_combine23(parts, h_prev, w_rel, w_root, b_rel, batch3):
    def body(parts_ref, hp_ref, wa_ref, wb_ref, b_ref, batch_ref, h_ref, pool_ref):
        i = pl.program_id(0)
        agg = parts_ref[0] + parts_ref[1]
        h = jnp.maximum(
            jnp.dot(agg, wa_ref[...], preferred_element_type=jnp.float32)
            + jnp.dot(hp_ref[...], wb_ref[...], preferred_element_type=jnp.float32)
            + b_ref[...],
            0.0,
        )
        h_ref[...] = h
        bt = batch_ref[0]
        gids = lax.broadcasted_iota(jnp.int32, (_G, _RB), 0)
        onehot = (gids == bt).astype(jnp.float32)
        _pool_accum(i, onehot, h, pool_ref)

    return pl.pallas_call(
        body,
        grid=(_NBLK,),
        in_specs=[
            pl.BlockSpec((_NC, _RB, _H), lambda i: (0, i, 0)),
            pl.BlockSpec((_RB, _H), lambda i: (i, 0)),
            pl.BlockSpec((_H, _H), lambda i: (0, 0)),
            pl.BlockSpec((_H, _H), lambda i: (0, 0)),
            pl.BlockSpec((1, _H), lambda i: (0, 0)),
            pl.BlockSpec((1, 1, _RB), lambda i: (i, 0, 0)),
        ],
        out_specs=[
            pl.BlockSpec((_RB, _H), lambda i: (i, 0)),
            pl.BlockSpec((_G, _H), lambda i: (0, 0)),
        ],
        out_shape=[
            jax.ShapeDtypeStruct((_N, _H), jnp.float32),
            jax.ShapeDtypeStruct((_G, _H), jnp.float32),
        ],
    )(parts, h_prev, w_rel, w_root, b_rel, batch3)


def _tc_final(p1, p2, p3, cnt, w1, b1, w2, b2):
    def body(p1_ref, p2_ref, p3_ref, cnt_ref, w1_ref, b1_ref, w2_ref, b2_ref, o_ref):
        inv = 1.0 / jnp.maximum(cnt_ref[...], 1.0)
        j = jnp.concatenate([p1_ref[...], p2_ref[...], p3_ref[...]], axis=1) * inv
        a = jnp.maximum(
            jnp.dot(j, w1_ref[...], preferred_element_type=jnp.float32) + b1_ref[...],
            0.0,
        )
        o_ref[...] = (
            jnp.dot(a, w2_ref[...], preferred_element_type=jnp.float32) + b2_ref[...]
        )

    return pl.pallas_call(
        body,
        out_shape=jax.ShapeDtypeStruct((_G, _C), jnp.float32),
    )(p1, p2, p3, cnt, w1, b1, w2, b2)


def kernel(x, edge_index, batch, W_rel1, b_rel1, W_root1, W_rel2, b_rel2, W_root2,
           W_rel3, b_rel3, W_root3, W1, b1, W2, b2):
    src = edge_index[0].astype(jnp.int32)
    dst = edge_index[1].astype(jnp.int32)
    pad = _EPAD - _E
    src3 = jnp.concatenate([src, jnp.zeros((pad,), jnp.int32)]).reshape(
        _NW, _NCHUNK, _CH)
    dst3 = jnp.concatenate([dst, jnp.full((pad,), _N, jnp.int32)]).reshape(
        _NW, _NCHUNK, _CH)
    batch3 = batch.astype(jnp.int32).reshape(_NBLK, 1, _RB)

    y1, r1 = _tc_pre(x, W_rel1, W_root1)
    parts1 = _sc_segment_sum(y1, src3, dst3)
    h1, pool1, cnt = _tc_combine1(parts1, r1, b_rel1.reshape(1, _H), batch3)
    parts2 = _sc_segment_sum(h1, src3, dst3)
    h2, pool2 = _tc_combine23(parts2, h1, W_rel2, W_root2,
                              b_rel2.reshape(1, _H), batch3)
    parts3 = _sc_segment_sum(h2, src3, dst3)
    _h3, pool3 = _tc_combine23(parts3, h2, W_rel3, W_root3,
                               b_rel3.reshape(1, _H), batch3)
    return _tc_final(pool1, pool2, pool3, cnt, W1, b1.reshape(1, _H),
                     W2, b2.reshape(1, _C))
